# baseline (speedup 1.0000x reference)
import functools

import jax
import jax.numpy as jnp
from jax.experimental import pallas as pl
from jax.experimental.pallas import tpu as pltpu

_ALPHA = 0.2


def _prep_kernel(x_ref, w_ref, atts_ref, attd_ref,
                 haug_ref, ht_ref, u_ref, p_ref, v_ref, q_ref, fd_ref):
    h = jnp.dot(x_ref[:], w_ref[:], preferred_element_type=jnp.float32)
    nh = h.shape[1]
    haug_ref[:, :nh] = h.astype(jnp.bfloat16)
    haug_ref[:, nh:] = jnp.ones_like(haug_ref[:, nh:])
    ht_ref[:] = h.T
    a_s_col = jax.lax.dot_general(h, atts_ref[:], (((1,), (1,)), ((), ())),
                                  preferred_element_type=jnp.float32)
    u_ref[:] = jnp.exp(a_s_col)
    p_ref[:] = jnp.exp(_ALPHA * a_s_col)
    ad = jax.lax.dot_general(attd_ref[:], h, (((1,), (1,)), ((), ())),
                             preferred_element_type=jnp.float32)
    a_s_row = jax.lax.dot_general(atts_ref[:], h, (((1,), (1,)), ((), ())),
                                  preferred_element_type=jnp.float32)
    v_ref[:] = jnp.exp(ad).astype(jnp.bfloat16)
    q_ref[:] = jnp.exp(_ALPHA * ad).astype(jnp.bfloat16)
    z = a_s_row + ad
    fd_ref[:] = jnp.maximum(jnp.exp(z), jnp.exp(_ALPHA * z))


def _gat_kernel(adja_ref, adjb_ref, hauga_ref, haugb_ref, ua_ref, ub_ref,
                pa_ref, pb_ref, v_ref, q_ref,
                ht_ref, fd_ref, bias_ref, out_ref, acc_ref, *, nh, nsteps):
    i = pl.program_id(0)

    @pl.when(i == 0)
    def _init():
        acc_ref[:] = jnp.zeros_like(acc_ref)

    v = v_ref[:]
    q = q_ref[:]
    for adj_ref, haug_ref, u_ref, p_ref in (
            (adja_ref, hauga_ref, ua_ref, pa_ref),
            (adjb_ref, haugb_ref, ub_ref, pb_ref)):
        u = u_ref[:].astype(jnp.bfloat16)
        p = p_ref[:].astype(jnp.bfloat16)
        m = jnp.maximum(u * v, p * q)
        P = adj_ref[0].astype(jnp.bfloat16) * m
        acc_ref[:] += jax.lax.dot_general(
            haug_ref[:], P, (((0,), (0,)), ((), ())),
            preferred_element_type=jnp.float32)

    @pl.when(i == nsteps - 1)
    def _finalize():
        fd = fd_ref[:]
        dn = acc_ref[nh:nh + 1, :] + fd + 1e-16
        o = (acc_ref[:nh, :] + fd * ht_ref[:]) / dn + bias_ref[:]
        out_ref[:] = jnp.where(o > 0, o, jnp.exp(o) - 1.0).T


def _pick_bs(n):
    for cand in (400, 320, 256, 200, 160, 128, 80, 40, 16, 8):
        if n % cand == 0:
            return cand
    return 8


def kernel(x, adj, W, att_src, att_dst, bias):
    n, nf = x.shape
    nh = W.shape[1]
    att_s2 = att_src.reshape(1, nh).astype(jnp.float32)
    att_d2 = att_dst.reshape(1, nh).astype(jnp.float32)
    bias_t = bias.reshape(nh, 1).astype(jnp.float32)

    haug, ht, u, p, v, q, fd = pl.pallas_call(
        _prep_kernel,
        out_shape=[
            jax.ShapeDtypeStruct((n, nh + 8), jnp.bfloat16),
            jax.ShapeDtypeStruct((nh, n), jnp.float32),
            jax.ShapeDtypeStruct((n, 1), jnp.float32),
            jax.ShapeDtypeStruct((n, 1), jnp.float32),
            jax.ShapeDtypeStruct((1, n), jnp.bfloat16),
            jax.ShapeDtypeStruct((1, n), jnp.bfloat16),
            jax.ShapeDtypeStruct((1, n), jnp.float32),
        ],
    )(x, W, att_s2, att_d2)

    bs = _pick_bs(n // 2)
    nsteps = (n // 2) // bs
    adj3 = adj.reshape(2, n // 2, n)
    out_t = pl.pallas_call(
        functools.partial(_gat_kernel, nh=nh, nsteps=nsteps),
        grid=(nsteps,),
        in_specs=[
            pl.BlockSpec((1, bs, n), lambda i: (0, i, 0)),
            pl.BlockSpec((1, bs, n), lambda i: (1, i, 0)),
            pl.BlockSpec((bs, nh + 8), lambda i: (i, 0)),
            pl.BlockSpec((bs, nh + 8),
                         lambda i, _ns=nsteps: (i + _ns, 0)),
            pl.BlockSpec((bs, 1), lambda i: (i, 0)),
            pl.BlockSpec((bs, 1), lambda i, _ns=nsteps: (i + _ns, 0)),
            pl.BlockSpec((bs, 1), lambda i: (i, 0)),
            pl.BlockSpec((bs, 1), lambda i, _ns=nsteps: (i + _ns, 0)),
            pl.BlockSpec((1, n), lambda i: (0, 0)),
            pl.BlockSpec((1, n), lambda i: (0, 0)),
            pl.BlockSpec((nh, n), lambda i: (0, 0)),
            pl.BlockSpec((1, n), lambda i: (0, 0)),
            pl.BlockSpec((nh, 1), lambda i: (0, 0)),
        ],
        out_specs=pl.BlockSpec((n, nh), lambda i: (0, 0)),
        out_shape=jax.ShapeDtypeStruct((n, nh), jnp.float32),
        scratch_shapes=[pltpu.VMEM((nh + 8, n), jnp.float32)],
    )(adj3, adj3, haug, haug, u, u, p, p, v, q, ht, fd, bias_t)
    return out_t


# speedup vs baseline: 1.0885x; 1.0885x over previous
import functools

import jax
import jax.numpy as jnp
from jax.experimental import pallas as pl
from jax.experimental.pallas import tpu as pltpu

_ALPHA = 0.2


def _gat_kernel(adj_ref, x_ref, w_ref, atts_ref, attd_ref, bias_ref,
                out_ref, acc_ref, haug_ref, ht_ref, v_ref, q_ref, fd_ref,
                *, bs, nh, nsteps):
    i = pl.program_id(0)

    @pl.when(i == 0)
    def _prep():
        acc_ref[:] = jnp.zeros_like(acc_ref)
        h = jnp.dot(x_ref[:], w_ref[:], preferred_element_type=jnp.float32)
        haug_ref[:, :nh] = h.astype(jnp.bfloat16)
        haug_ref[:, nh:] = jnp.ones(haug_ref[:, nh:].shape, jnp.bfloat16)
        ht_ref[:] = h.T
        ad = jax.lax.dot_general(attd_ref[:], h, (((1,), (1,)), ((), ())),
                                 preferred_element_type=jnp.float32)
        a_s = jax.lax.dot_general(atts_ref[:], h, (((1,), (1,)), ((), ())),
                                  preferred_element_type=jnp.float32)
        v_ref[:] = jnp.exp(ad).astype(jnp.bfloat16)
        q_ref[:] = jnp.exp(_ALPHA * ad).astype(jnp.bfloat16)
        z = a_s + ad
        fd_ref[:] = jnp.maximum(jnp.exp(z), jnp.exp(_ALPHA * z))

    haug = haug_ref[pl.ds(i * bs, bs), :]
    a_s = jax.lax.dot_general(haug[:, :nh].astype(jnp.float32), atts_ref[:],
                              (((1,), (1,)), ((), ())),
                              preferred_element_type=jnp.float32)
    u = jnp.exp(a_s).astype(jnp.bfloat16)
    p = jnp.exp(_ALPHA * a_s).astype(jnp.bfloat16)
    m = jnp.maximum(u * v_ref[:], p * q_ref[:])
    P = adj_ref[:].astype(jnp.bfloat16) * m

    acc_ref[:] += jax.lax.dot_general(haug, P, (((0,), (0,)), ((), ())),
                                      preferred_element_type=jnp.float32)

    @pl.when(i == nsteps - 1)
    def _finalize():
        fd = fd_ref[:]
        dn = acc_ref[nh:nh + 1, :] + fd + 1e-16
        o = (acc_ref[:nh, :] + fd * ht_ref[:]) / dn + bias_ref[:]
        out_ref[:] = jnp.where(o > 0, o, jnp.exp(o) - 1.0).T


def _pick_bs(n):
    for cand in (200, 160, 128, 80, 40, 16, 8):
        if n % cand == 0:
            return cand
    return 8


def kernel(x, adj, W, att_src, att_dst, bias):
    n, nf = x.shape
    nh = W.shape[1]
    att_s2 = att_src.reshape(1, nh).astype(jnp.float32)
    att_d2 = att_dst.reshape(1, nh).astype(jnp.float32)
    bias_t = bias.reshape(nh, 1).astype(jnp.float32)

    bs = _pick_bs(n)
    nsteps = n // bs
    out = pl.pallas_call(
        functools.partial(_gat_kernel, bs=bs, nh=nh, nsteps=nsteps),
        grid=(nsteps,),
        in_specs=[
            pl.BlockSpec((bs, n), lambda i: (i, 0)),
            pl.BlockSpec((n, nf), lambda i: (0, 0)),
            pl.BlockSpec((nf, nh), lambda i: (0, 0)),
            pl.BlockSpec((1, nh), lambda i: (0, 0)),
            pl.BlockSpec((1, nh), lambda i: (0, 0)),
            pl.BlockSpec((nh, 1), lambda i: (0, 0)),
        ],
        out_specs=pl.BlockSpec((n, nh), lambda i: (0, 0)),
        out_shape=jax.ShapeDtypeStruct((n, nh), jnp.float32),
        scratch_shapes=[
            pltpu.VMEM((nh + 8, n), jnp.float32),
            pltpu.VMEM((n, nh + 8), jnp.bfloat16),
            pltpu.VMEM((nh, n), jnp.float32),
            pltpu.VMEM((1, n), jnp.bfloat16),
            pltpu.VMEM((1, n), jnp.bfloat16),
            pltpu.VMEM((1, n), jnp.float32),
        ],
    )(adj, x, W, att_s2, att_d2, bias_t)
    return out
